# Initial kernel scaffold; baseline (speedup 1.0000x reference)
import jax
import jax.numpy as jnp
from jax.experimental import pallas as pl


def kernel(X, positions_table, chroms, chrom_table):
    raise NotImplementedError("write your pallas kernel here")


# speedup vs baseline: 1.6507x; 1.6507x over previous
import functools

import jax
import jax.numpy as jnp
from jax import lax
from jax.experimental import pallas as pl
from jax.experimental.pallas import tpu as pltpu
from jax.experimental.pallas import tpu_sc as plsc

D_MODEL = 64
MIN_FREQ = 1e-4


def _make_sc_gather(n_total):
    info = plsc.get_sparse_core_info()
    nw = info.num_cores * info.num_subcores
    assert n_total % nw == 0
    chunk = n_total // nw
    assert chunk % 8 == 0

    mesh = plsc.VectorSubcoreMesh(core_axis_name="c", subcore_axis_name="s")

    @functools.partial(
        pl.kernel,
        mesh=mesh,
        out_type=[
            jax.ShapeDtypeStruct((n_total,), jnp.float32),
            jax.ShapeDtypeStruct((n_total,), jnp.int32),
        ],
        scratch_types=[
            pltpu.VMEM((chunk,), jnp.int32),
            pltpu.VMEM((chunk,), jnp.float32),
            pltpu.VMEM((chunk,), jnp.int32),
            pltpu.SemaphoreType.DMA,
            pltpu.SemaphoreType.DMA,
        ],
    )
    def sc_gather(x_hbm, ptab_hbm, ctab_hbm, pos_hbm, cid_hbm,
                  idx_v, pos_v, cid_v, sem_p, sem_c):
        wid = lax.axis_index("s") * info.num_cores + lax.axis_index("c")
        base = wid * chunk
        pltpu.sync_copy(x_hbm.at[pl.ds(base, chunk)], idx_v)
        cp_p = pltpu.async_copy(ptab_hbm.at[idx_v], pos_v, sem_p)
        cp_c = pltpu.async_copy(ctab_hbm.at[idx_v], cid_v, sem_c)
        cp_p.wait()
        cp_c.wait()
        pltpu.sync_copy(pos_v, pos_hbm.at[pl.ds(base, chunk)])
        pltpu.sync_copy(cid_v, cid_hbm.at[pl.ds(base, chunk)])

    return sc_gather


def _encode_body(ba, n_chroms, pos_ref, cid_ref, freqs_ref, tab_ref, out_ref):
    ll = pos_ref.shape[0]
    tab = tab_ref[...]
    freqs = freqs_ref[...]
    pos_blk = pos_ref[...]
    cidf_blk = cid_ref[...].astype(jnp.float32)
    ci = lax.broadcasted_iota(jnp.int32, (ll, n_chroms), 1).astype(jnp.float32)
    parity = lax.broadcasted_iota(jnp.int32, (ll, D_MODEL), 1) % 2 == 0

    def step(a, _):
        e_row = (lax.broadcasted_iota(jnp.int32, (1, ba), 1) == a
                 ).astype(jnp.float32)
        p = jnp.sum(pos_blk * e_row, axis=1, keepdims=True)
        cf = jnp.sum(cidf_blk * e_row, axis=1, keepdims=True)
        ang = p * freqs
        oh = (cf == ci).astype(jnp.float32)
        emb = lax.dot_general(oh, tab, (((1,), (0,)), ((), ())),
                              preferred_element_type=jnp.float32)
        enc = jnp.where(parity, jnp.sin(ang), jnp.cos(ang))
        out_ref[pl.ds(a, 1)] = (emb + enc)[None]
        return 0

    lax.fori_loop(0, ba, step, 0)


def _tc_encode(pos_t, cid_t, freqs, chrom_table, ba):
    ll, aa = pos_t.shape
    n_chroms = chrom_table.shape[0]
    assert aa % ba == 0
    grid = (aa // ba,)
    return pl.pallas_call(
        functools.partial(_encode_body, ba, n_chroms),
        grid=grid,
        in_specs=[
            pl.BlockSpec((ll, ba), lambda i: (0, i)),
            pl.BlockSpec((ll, ba), lambda i: (0, i)),
            pl.BlockSpec((1, D_MODEL), lambda i: (0, 0)),
            pl.BlockSpec((n_chroms, D_MODEL), lambda i: (0, 0)),
        ],
        out_specs=pl.BlockSpec((ba, ll, D_MODEL), lambda i: (i, 0, 0)),
        out_shape=jax.ShapeDtypeStruct((aa, ll, D_MODEL), jnp.float32),
    )(pos_t, cid_t, freqs, chrom_table)


def _freqs_row():
    i = jnp.arange(D_MODEL)
    return (jnp.asarray(MIN_FREQ, jnp.float32)
            ** (2.0 * (i // 2).astype(jnp.float32) / D_MODEL)).reshape(1, D_MODEL)


def kernel(X, positions_table, chroms, chrom_table):
    a, l = X.shape
    n = a * l
    xf = X.reshape(n).astype(jnp.int32)
    pos_flat, cid_flat = _make_sc_gather(n)(
        xf, positions_table.astype(jnp.float32), chroms.astype(jnp.int32))
    pos_t = pos_flat.reshape(a, l).T
    cid_t = cid_flat.reshape(a, l).T
    out = _tc_encode(pos_t, cid_t, _freqs_row(), chrom_table, ba=128)
    return out


# speedup vs baseline: 2.8705x; 1.7390x over previous
import functools

import jax
import jax.numpy as jnp
from jax import lax
from jax.experimental import pallas as pl
from jax.experimental.pallas import tpu as pltpu
from jax.experimental.pallas import tpu_sc as plsc

D_MODEL = 64
MIN_FREQ = 1e-4


def _make_sc_gather(n_total):
    info = plsc.get_sparse_core_info()
    nw = info.num_cores * info.num_subcores
    assert n_total % nw == 0
    chunk = n_total // nw
    assert chunk % 8 == 0

    mesh = plsc.VectorSubcoreMesh(core_axis_name="c", subcore_axis_name="s")

    @functools.partial(
        pl.kernel,
        mesh=mesh,
        out_type=[
            jax.ShapeDtypeStruct((n_total,), jnp.float32),
            jax.ShapeDtypeStruct((n_total,), jnp.int32),
        ],
        scratch_types=[
            pltpu.VMEM((chunk,), jnp.int32),
            pltpu.VMEM((chunk,), jnp.float32),
            pltpu.VMEM((chunk,), jnp.int32),
            pltpu.SemaphoreType.DMA,
            pltpu.SemaphoreType.DMA,
        ],
    )
    def sc_gather(x_hbm, ptab_hbm, ctab_hbm, pos_hbm, cid_hbm,
                  idx_v, pos_v, cid_v, sem_p, sem_c):
        wid = lax.axis_index("s") * info.num_cores + lax.axis_index("c")
        base = wid * chunk
        pltpu.sync_copy(x_hbm.at[pl.ds(base, chunk)], idx_v)
        cp_p = pltpu.async_copy(ptab_hbm.at[idx_v], pos_v, sem_p)
        cp_c = pltpu.async_copy(ctab_hbm.at[idx_v], cid_v, sem_c)
        cp_p.wait()
        cp_c.wait()
        pltpu.sync_copy(pos_v, pos_hbm.at[pl.ds(base, chunk)])
        pltpu.sync_copy(cid_v, cid_hbm.at[pl.ds(base, chunk)])

    return sc_gather


def _encode_body(ba, n_chroms, pos_ref, cid_ref, freqs2_ref, tab2_ref, out_ref):
    ll = pos_ref.shape[0]
    tab2 = tab2_ref[...]
    freqs2 = freqs2_ref[...]
    pos_blk = pos_ref[...]
    cidf_blk = cid_ref[...].astype(jnp.float32)
    parity = lax.broadcasted_iota(jnp.int32, (ll, 2 * D_MODEL), 1) % 2 == 0
    half = lax.broadcasted_iota(jnp.int32, (1, 2 * D_MODEL), 1) < D_MODEL
    halfk = lax.broadcasted_iota(jnp.int32, (1, 2 * n_chroms), 1) < n_chroms
    k48 = (lax.broadcasted_iota(jnp.int32, (1, 2 * n_chroms), 1) % n_chroms
           ).astype(jnp.float32)
    ncols = 8

    def step(j, _):
        e8 = (lax.broadcasted_iota(jnp.int32, (ba, ncols), 0)
              == j * ncols + lax.broadcasted_iota(jnp.int32, (ba, ncols), 1)
              ).astype(jnp.float32)
        p8 = lax.dot_general(pos_blk, e8, (((1,), (0,)), ((), ())),
                             precision=lax.Precision.HIGHEST,
                             preferred_element_type=jnp.float32)
        c8 = lax.dot_general(cidf_blk, e8, (((1,), (0,)), ((), ())),
                             preferred_element_type=jnp.float32)
        for g in range(ncols // 2):
            p0 = p8[:, 2 * g:2 * g + 1]
            p1 = p8[:, 2 * g + 1:2 * g + 2]
            p2 = jnp.where(half, p0, p1)
            ang2 = p2 * freqs2
            c0 = c8[:, 2 * g:2 * g + 1]
            c1 = c8[:, 2 * g + 1:2 * g + 2]
            csel = jnp.where(halfk, c0, c1)
            oh2 = (csel == k48).astype(jnp.float32)
            emb2 = lax.dot_general(oh2, tab2, (((1,), (0,)), ((), ())),
                                   preferred_element_type=jnp.float32)
            o2 = emb2 + jnp.where(parity, jnp.sin(ang2), jnp.cos(ang2))
            a_idx = j * ncols + 2 * g
            out_ref[pl.ds(a_idx, 1)] = o2[:, :D_MODEL][None]
            out_ref[pl.ds(a_idx + 1, 1)] = o2[:, D_MODEL:][None]
        return 0

    lax.fori_loop(0, ba // ncols, step, 0)


def _tc_encode(pos_t, cid_t, freqs2, tab2, ba):
    ll, aa = pos_t.shape
    n_chroms = tab2.shape[0] // 2
    assert aa % ba == 0
    grid = (aa // ba,)
    return pl.pallas_call(
        functools.partial(_encode_body, ba, n_chroms),
        grid=grid,
        in_specs=[
            pl.BlockSpec((ll, ba), lambda i: (0, i)),
            pl.BlockSpec((ll, ba), lambda i: (0, i)),
            pl.BlockSpec((1, 2 * D_MODEL), lambda i: (0, 0)),
            pl.BlockSpec((2 * n_chroms, 2 * D_MODEL), lambda i: (0, 0)),
        ],
        out_specs=pl.BlockSpec((ba, ll, D_MODEL), lambda i: (i, 0, 0)),
        out_shape=jax.ShapeDtypeStruct((aa, ll, D_MODEL), jnp.float32),
    )(pos_t, cid_t, freqs2, tab2)


def _freqs_row():
    i = jnp.arange(D_MODEL)
    return (jnp.asarray(MIN_FREQ, jnp.float32)
            ** (2.0 * (i // 2).astype(jnp.float32) / D_MODEL)).reshape(1, D_MODEL)


def kernel(X, positions_table, chroms, chrom_table):
    a, l = X.shape
    n = a * l
    xf = X.reshape(n).astype(jnp.int32)
    pos_flat, cid_flat = _make_sc_gather(n)(
        xf, positions_table.astype(jnp.float32), chroms.astype(jnp.int32))
    pos_t = pos_flat.reshape(a, l).T
    cid_t = cid_flat.reshape(a, l).T
    fr = _freqs_row()
    freqs2 = jnp.concatenate([fr, fr], axis=1)
    nc, d = chrom_table.shape
    z = jnp.zeros((nc, d), jnp.float32)
    tab2 = jnp.concatenate(
        [jnp.concatenate([chrom_table, z], axis=1),
         jnp.concatenate([z, chrom_table], axis=1)], axis=0)
    out = _tc_encode(pos_t, cid_t, freqs2, tab2, ba=128)
    return out


# speedup vs baseline: 3.5990x; 1.2538x over previous
import functools

import jax
import jax.numpy as jnp
import numpy as np
from jax import lax
from jax.experimental import pallas as pl
from jax.experimental.pallas import tpu as pltpu
from jax.experimental.pallas import tpu_sc as plsc

D_MODEL = 64
MIN_FREQ = 1e-4


def _hi14(x):
    f = np.float32(x)
    bits = f.view(np.int32) & np.int32(~0x3FF)
    return float(np.int32(bits).view(np.float32))


_TWO_OVER_PI = float(np.float32(2.0 / np.pi))
_MAGIC = 12582912.0
_T_FULL = 1024.0 * (np.pi / 2.0)
_TA = _hi14(_T_FULL)
_TB = _hi14(_T_FULL - _TA)
_TC = float(np.float32(_T_FULL - _TA - _TB))
_C_FULL = np.pi / 2.0
_CA = _hi14(_C_FULL)
_CB = _hi14(_C_FULL - _CA)
_CC = float(np.float32(_C_FULL - _CA - _CB))
_S3, _S5, _S7, _S9 = (-1.0 / 6.0, 1.0 / 120.0, -1.0 / 5040.0, 1.0 / 362880.0)
_C2, _C4, _C6, _C8 = (-0.5, 1.0 / 24.0, -1.0 / 720.0, 1.0 / 40320.0)


def _sincos_lanes(ang, par):
    w = ang * _TWO_OVER_PI
    qf = (w + _MAGIC) - _MAGIC
    qi = qf.astype(jnp.int32)
    qh = (qi >> 10).astype(jnp.float32)
    ql = (qi & 1023).astype(jnp.float32)
    r = ang - qh * _TA
    r = r - ql * _CA
    r = r - qh * _TB
    r = r - ql * _CB
    r = r - qh * _TC
    r = r - ql * _CC
    m = (qi + par) & 3
    r2 = r * r
    ps = r * (1.0 + r2 * (_S3 + r2 * (_S5 + r2 * (_S7 + r2 * _S9))))
    pc = 1.0 + r2 * (_C2 + r2 * (_C4 + r2 * (_C6 + r2 * _C8)))
    val = jnp.where((m & 1) == 1, pc, ps)
    sgn = (m & 2) << 30
    bits = lax.bitcast_convert_type(val, jnp.int32) ^ sgn
    return lax.bitcast_convert_type(bits, jnp.float32)


def _make_sc_gather(n_total):
    info = plsc.get_sparse_core_info()
    nw = info.num_cores * info.num_subcores
    assert n_total % nw == 0
    chunk = n_total // nw
    assert chunk % 8 == 0

    mesh = plsc.VectorSubcoreMesh(core_axis_name="c", subcore_axis_name="s")

    @functools.partial(
        pl.kernel,
        mesh=mesh,
        out_type=[
            jax.ShapeDtypeStruct((n_total,), jnp.float32),
            jax.ShapeDtypeStruct((n_total,), jnp.int32),
        ],
        scratch_types=[
            pltpu.VMEM((chunk,), jnp.int32),
            pltpu.VMEM((chunk,), jnp.float32),
            pltpu.VMEM((chunk,), jnp.int32),
            pltpu.SemaphoreType.DMA,
            pltpu.SemaphoreType.DMA,
        ],
    )
    def sc_gather(x_hbm, ptab_hbm, ctab_hbm, pos_hbm, cid_hbm,
                  idx_v, pos_v, cid_v, sem_p, sem_c):
        wid = lax.axis_index("s") * info.num_cores + lax.axis_index("c")
        base = wid * chunk
        pltpu.sync_copy(x_hbm.at[pl.ds(base, chunk)], idx_v)
        cp_p = pltpu.async_copy(ptab_hbm.at[idx_v], pos_v, sem_p)
        cp_c = pltpu.async_copy(ctab_hbm.at[idx_v], cid_v, sem_c)
        cp_p.wait()
        cp_c.wait()
        pltpu.sync_copy(pos_v, pos_hbm.at[pl.ds(base, chunk)])
        pltpu.sync_copy(cid_v, cid_hbm.at[pl.ds(base, chunk)])

    return sc_gather


def _encode_body(ba, n_chroms, pos_ref, cid_ref, freqs2_ref, tab2_ref, out_ref):
    ll = pos_ref.shape[0]
    tab2 = tab2_ref[...]
    freqs2 = freqs2_ref[...]
    pos_blk = pos_ref[...]
    cidf_blk = cid_ref[...].astype(jnp.float32)
    parity = lax.broadcasted_iota(jnp.int32, (1, 2 * D_MODEL), 1) & 1
    half = lax.broadcasted_iota(jnp.int32, (1, 2 * D_MODEL), 1) < D_MODEL
    halfk = lax.broadcasted_iota(jnp.int32, (1, 2 * n_chroms), 1) < n_chroms
    k48 = (lax.broadcasted_iota(jnp.int32, (1, 2 * n_chroms), 1) % n_chroms
           ).astype(jnp.float32)
    ncols = 8

    def step(j, _):
        e8 = (lax.broadcasted_iota(jnp.int32, (ba, ncols), 0)
              == j * ncols + lax.broadcasted_iota(jnp.int32, (ba, ncols), 1)
              ).astype(jnp.float32)
        p8 = lax.dot_general(pos_blk, e8, (((1,), (0,)), ((), ())),
                             precision=lax.Precision.HIGHEST,
                             preferred_element_type=jnp.float32)
        c8 = lax.dot_general(cidf_blk, e8, (((1,), (0,)), ((), ())),
                             preferred_element_type=jnp.float32)
        for g in range(ncols // 2):
            p0 = p8[:, 2 * g:2 * g + 1]
            p1 = p8[:, 2 * g + 1:2 * g + 2]
            p2 = jnp.where(half, p0, p1)
            ang2 = p2 * freqs2
            c0 = c8[:, 2 * g:2 * g + 1]
            c1 = c8[:, 2 * g + 1:2 * g + 2]
            csel = jnp.where(halfk, c0, c1)
            oh2 = (csel == k48).astype(jnp.float32)
            emb2 = lax.dot_general(oh2, tab2, (((1,), (0,)), ((), ())),
                                   preferred_element_type=jnp.float32)
            o2 = emb2 + _sincos_lanes(ang2, parity)
            a_idx = j * ncols + 2 * g
            out_ref[pl.ds(a_idx, 1)] = o2[:, :D_MODEL][None]
            out_ref[pl.ds(a_idx + 1, 1)] = o2[:, D_MODEL:][None]
        return 0

    lax.fori_loop(0, ba // ncols, step, 0)


def _tc_encode(pos_t, cid_t, freqs2, tab2, ba):
    ll, aa = pos_t.shape
    n_chroms = tab2.shape[0] // 2
    assert aa % ba == 0
    grid = (aa // ba,)
    return pl.pallas_call(
        functools.partial(_encode_body, ba, n_chroms),
        grid=grid,
        in_specs=[
            pl.BlockSpec((ll, ba), lambda i: (0, i)),
            pl.BlockSpec((ll, ba), lambda i: (0, i)),
            pl.BlockSpec((1, 2 * D_MODEL), lambda i: (0, 0)),
            pl.BlockSpec((2 * n_chroms, 2 * D_MODEL), lambda i: (0, 0)),
        ],
        out_specs=pl.BlockSpec((ba, ll, D_MODEL), lambda i: (i, 0, 0)),
        out_shape=jax.ShapeDtypeStruct((aa, ll, D_MODEL), jnp.float32),
    )(pos_t, cid_t, freqs2, tab2)


def _freqs_row():
    i = jnp.arange(D_MODEL)
    return (jnp.asarray(MIN_FREQ, jnp.float32)
            ** (2.0 * (i // 2).astype(jnp.float32) / D_MODEL)).reshape(1, D_MODEL)


def kernel(X, positions_table, chroms, chrom_table):
    a, l = X.shape
    n = a * l
    xf = X.reshape(n).astype(jnp.int32)
    pos_flat, cid_flat = _make_sc_gather(n)(
        xf, positions_table.astype(jnp.float32), chroms.astype(jnp.int32))
    pos_t = pos_flat.reshape(a, l).T
    cid_t = cid_flat.reshape(a, l).T
    fr = _freqs_row()
    freqs2 = jnp.concatenate([fr, fr], axis=1)
    nc, d = chrom_table.shape
    z = jnp.zeros((nc, d), jnp.float32)
    tab2 = jnp.concatenate(
        [jnp.concatenate([chrom_table, z], axis=1),
         jnp.concatenate([z, chrom_table], axis=1)], axis=0)
    out = _tc_encode(pos_t, cid_t, freqs2, tab2, ba=128)
    return out


# speedup vs baseline: 4.5688x; 1.2694x over previous
import functools

import jax
import jax.numpy as jnp
import numpy as np
from jax import lax
from jax.experimental import pallas as pl
from jax.experimental.pallas import tpu as pltpu
from jax.experimental.pallas import tpu_sc as plsc

D_MODEL = 64
MIN_FREQ = 1e-4


def _hi14(x):
    f = np.float32(x)
    bits = f.view(np.int32) & np.int32(~0x3FF)
    return float(np.int32(bits).view(np.float32))


_TWO_OVER_PI = float(np.float32(2.0 / np.pi))
_MAGIC = 12582912.0
_T_FULL = 1024.0 * (np.pi / 2.0)
_TA = _hi14(_T_FULL)
_TB = _hi14(_T_FULL - _TA)
_TC = float(np.float32(_T_FULL - _TA - _TB))
_C_FULL = np.pi / 2.0
_CA = _hi14(_C_FULL)
_CB = _hi14(_C_FULL - _CA)
_CC = float(np.float32(_C_FULL - _CA - _CB))
_S3, _S5, _S7, _S9 = (-1.0 / 6.0, 1.0 / 120.0, -1.0 / 5040.0, 1.0 / 362880.0)
_C2, _C4, _C6, _C8 = (-0.5, 1.0 / 24.0, -1.0 / 720.0, 1.0 / 40320.0)


def _sincos_lanes(ang, par):
    w = ang * _TWO_OVER_PI
    qf = (w + _MAGIC) - _MAGIC
    qi = qf.astype(jnp.int32)
    qh = (qi >> 10).astype(jnp.float32)
    ql = (qi & 1023).astype(jnp.float32)
    r = ang - qh * _TA
    r = r - ql * _CA
    r = r - qh * _TB
    r = r - ql * _CB
    r = r - qh * _TC
    r = r - ql * _CC
    m = (qi + par) & 3
    r2 = r * r
    ps = r * (1.0 + r2 * (_S3 + r2 * (_S5 + r2 * (_S7 + r2 * _S9))))
    pc = 1.0 + r2 * (_C2 + r2 * (_C4 + r2 * (_C6 + r2 * _C8)))
    val = jnp.where((m & 1) == 1, pc, ps)
    sgn = (m & 2) << 30
    bits = lax.bitcast_convert_type(val, jnp.int32) ^ sgn
    return lax.bitcast_convert_type(bits, jnp.float32)


def _make_sc_gather(n_total):
    info = plsc.get_sparse_core_info()
    nw = info.num_cores * info.num_subcores
    assert n_total % nw == 0
    chunk = n_total // nw
    assert chunk % 8 == 0

    mesh = plsc.VectorSubcoreMesh(core_axis_name="c", subcore_axis_name="s")

    @functools.partial(
        pl.kernel,
        mesh=mesh,
        out_type=[
            jax.ShapeDtypeStruct((n_total,), jnp.float32),
            jax.ShapeDtypeStruct((n_total,), jnp.int32),
        ],
        scratch_types=[
            pltpu.VMEM((chunk,), jnp.int32),
            pltpu.VMEM((chunk,), jnp.float32),
            pltpu.VMEM((chunk,), jnp.int32),
            pltpu.SemaphoreType.DMA,
            pltpu.SemaphoreType.DMA,
        ],
    )
    def sc_gather(x_hbm, ptab_hbm, ctab_hbm, pos_hbm, cid_hbm,
                  idx_v, pos_v, cid_v, sem_p, sem_c):
        wid = lax.axis_index("s") * info.num_cores + lax.axis_index("c")
        base = wid * chunk
        pltpu.sync_copy(x_hbm.at[pl.ds(base, chunk)], idx_v)
        cp_p = pltpu.async_copy(ptab_hbm.at[idx_v], pos_v, sem_p)
        cp_c = pltpu.async_copy(ctab_hbm.at[idx_v], cid_v, sem_c)
        cp_p.wait()
        cp_c.wait()
        pltpu.sync_copy(pos_v, pos_hbm.at[pl.ds(base, chunk)])
        pltpu.sync_copy(cid_v, cid_hbm.at[pl.ds(base, chunk)])

    return sc_gather


def _encode_body(ba, n_chroms, pos_ref, cid_ref, freqs2_ref, tab2_ref, out_ref):
    ll = pos_ref.shape[0]
    tab2 = tab2_ref[...]
    freqs2 = freqs2_ref[...]
    pos_blk = pos_ref[...]
    cidf_blk = cid_ref[...].astype(jnp.float32)
    parity = lax.broadcasted_iota(jnp.int32, (1, 2 * D_MODEL), 1) & 1
    half = lax.broadcasted_iota(jnp.int32, (1, 2 * D_MODEL), 1) < D_MODEL
    halfk = lax.broadcasted_iota(jnp.int32, (1, 2 * n_chroms), 1) < n_chroms
    k48 = (lax.broadcasted_iota(jnp.int32, (1, 2 * n_chroms), 1) % n_chroms
           ).astype(jnp.float32)
    ncols = 16

    def step(j, _):
        e8 = (lax.broadcasted_iota(jnp.int32, (ba, ncols), 0)
              == j * ncols + lax.broadcasted_iota(jnp.int32, (ba, ncols), 1)
              ).astype(jnp.float32)
        p8 = lax.dot_general(pos_blk, e8, (((1,), (0,)), ((), ())),
                             precision=lax.Precision.HIGHEST,
                             preferred_element_type=jnp.float32)
        c8 = lax.dot_general(cidf_blk, e8, (((1,), (0,)), ((), ())),
                             preferred_element_type=jnp.float32)
        for g in range(ncols // 2):
            p0 = p8[:, 2 * g:2 * g + 1]
            p1 = p8[:, 2 * g + 1:2 * g + 2]
            p2 = jnp.where(half, p0, p1)
            ang2 = p2 * freqs2
            c0 = c8[:, 2 * g:2 * g + 1]
            c1 = c8[:, 2 * g + 1:2 * g + 2]
            csel = jnp.where(halfk, c0, c1)
            oh2 = (csel == k48).astype(jnp.float32)
            emb2 = lax.dot_general(oh2, tab2, (((1,), (0,)), ((), ())),
                                   preferred_element_type=jnp.float32)
            o2 = emb2 + _sincos_lanes(ang2, parity)
            a_idx = j * ncols + 2 * g
            out_ref[pl.ds(a_idx, 1)] = o2[:, :D_MODEL][None]
            out_ref[pl.ds(a_idx + 1, 1)] = o2[:, D_MODEL:][None]
        return 0

    lax.fori_loop(0, ba // ncols, step, 0)


def _tc_encode(pos_t, cid_t, freqs2, tab2, ba):
    ll, aa = pos_t.shape
    n_chroms = tab2.shape[0] // 2
    assert aa % ba == 0
    grid = (aa // ba,)
    return pl.pallas_call(
        functools.partial(_encode_body, ba, n_chroms),
        grid=grid,
        in_specs=[
            pl.BlockSpec((ll, ba), lambda i: (0, i)),
            pl.BlockSpec((ll, ba), lambda i: (0, i)),
            pl.BlockSpec((1, 2 * D_MODEL), lambda i: (0, 0)),
            pl.BlockSpec((2 * n_chroms, 2 * D_MODEL), lambda i: (0, 0)),
        ],
        out_specs=pl.BlockSpec((ba, ll, D_MODEL), lambda i: (i, 0, 0)),
        out_shape=jax.ShapeDtypeStruct((aa, ll, D_MODEL), jnp.float32),
    )(pos_t, cid_t, freqs2, tab2)


def _freqs_row():
    i = jnp.arange(D_MODEL)
    return (jnp.asarray(MIN_FREQ, jnp.float32)
            ** (2.0 * (i // 2).astype(jnp.float32) / D_MODEL)).reshape(1, D_MODEL)


def kernel(X, positions_table, chroms, chrom_table):
    a, l = X.shape
    n = a * l
    xf = X.reshape(n).astype(jnp.int32)
    pos_flat, cid_flat = _make_sc_gather(n)(
        xf, positions_table.astype(jnp.float32), chroms.astype(jnp.int32))
    pos_t = pos_flat.reshape(a, l).T
    cid_t = cid_flat.reshape(a, l).T
    fr = _freqs_row()
    freqs2 = jnp.concatenate([fr, fr], axis=1)
    nc, d = chrom_table.shape
    z = jnp.zeros((nc, d), jnp.float32)
    tab2 = jnp.concatenate(
        [jnp.concatenate([chrom_table, z], axis=1),
         jnp.concatenate([z, chrom_table], axis=1)], axis=0)
    out = _tc_encode(pos_t, cid_t, freqs2, tab2, ba=128)
    return out


# speedup vs baseline: 4.6812x; 1.0246x over previous
import functools

import jax
import jax.numpy as jnp
import numpy as np
from jax import lax
from jax.experimental import pallas as pl
from jax.experimental.pallas import tpu as pltpu
from jax.experimental.pallas import tpu_sc as plsc

D_MODEL = 64
MIN_FREQ = 1e-4


def _hi14(x):
    f = np.float32(x)
    bits = f.view(np.int32) & np.int32(~0x3FF)
    return float(np.int32(bits).view(np.float32))


_TWO_OVER_PI = float(np.float32(2.0 / np.pi))
_MAGIC = 12582912.0
_T_FULL = 1024.0 * (np.pi / 2.0)
_TA = _hi14(_T_FULL)
_TB = _hi14(_T_FULL - _TA)
_TC = float(np.float32(_T_FULL - _TA - _TB))
_C_FULL = np.pi / 2.0
_CA = _hi14(_C_FULL)
_CB = _hi14(_C_FULL - _CA)
_CC = float(np.float32(_C_FULL - _CA - _CB))
_S3, _S5, _S7, _S9 = (-1.0 / 6.0, 1.0 / 120.0, -1.0 / 5040.0, 1.0 / 362880.0)
_C2, _C4, _C6, _C8 = (-0.5, 1.0 / 24.0, -1.0 / 720.0, 1.0 / 40320.0)


def _sincos_lanes(ang, par):
    w = ang * _TWO_OVER_PI
    qf = (w + _MAGIC) - _MAGIC
    qi = qf.astype(jnp.int32)
    qh = (qi >> 10).astype(jnp.float32)
    ql = (qi & 1023).astype(jnp.float32)
    r = ang - qh * _TA
    r = r - ql * _CA
    r = r - qh * _TB
    r = r - ql * _CB
    r = r - qh * _TC
    r = r - ql * _CC
    m = (qi + par) & 3
    r2 = r * r
    ps = r * (1.0 + r2 * (_S3 + r2 * (_S5 + r2 * (_S7 + r2 * _S9))))
    pc = 1.0 + r2 * (_C2 + r2 * (_C4 + r2 * (_C6 + r2 * _C8)))
    val = jnp.where((m & 1) == 1, pc, ps)
    sgn = (m & 2) << 30
    bits = lax.bitcast_convert_type(val, jnp.int32) ^ sgn
    return lax.bitcast_convert_type(bits, jnp.float32)


def _make_sc_gather(n_total):
    info = plsc.get_sparse_core_info()
    nw = info.num_cores * info.num_subcores
    assert n_total % nw == 0
    chunk = n_total // nw
    assert chunk % 8 == 0

    mesh = plsc.VectorSubcoreMesh(core_axis_name="c", subcore_axis_name="s")

    @functools.partial(
        pl.kernel,
        mesh=mesh,
        out_type=[
            jax.ShapeDtypeStruct((n_total,), jnp.float32),
            jax.ShapeDtypeStruct((n_total,), jnp.int32),
        ],
        scratch_types=[
            pltpu.VMEM((chunk,), jnp.int32),
            pltpu.VMEM((chunk,), jnp.float32),
            pltpu.VMEM((chunk,), jnp.int32),
            pltpu.SemaphoreType.DMA,
            pltpu.SemaphoreType.DMA,
        ],
    )
    def sc_gather(x_hbm, ptab_hbm, ctab_hbm, pos_hbm, cid_hbm,
                  idx_v, pos_v, cid_v, sem_p, sem_c):
        wid = lax.axis_index("s") * info.num_cores + lax.axis_index("c")
        base = wid * chunk
        pltpu.sync_copy(x_hbm.at[pl.ds(base, chunk)], idx_v)
        cp_p = pltpu.async_copy(ptab_hbm.at[idx_v], pos_v, sem_p)
        cp_c = pltpu.async_copy(ctab_hbm.at[idx_v], cid_v, sem_c)
        cp_p.wait()
        cp_c.wait()
        pltpu.sync_copy(pos_v, pos_hbm.at[pl.ds(base, chunk)])
        pltpu.sync_copy(cid_v, cid_hbm.at[pl.ds(base, chunk)])

    return sc_gather


def _encode_body(ba, n_chroms, pos_ref, cid_ref, freqs2_ref, tab2_ref, out_ref):
    ll = pos_ref.shape[0]
    tab2 = tab2_ref[...]
    freqs2 = freqs2_ref[...]
    pos_blk = pos_ref[...]
    cidf_blk = cid_ref[...].astype(jnp.float32)
    parity = lax.broadcasted_iota(jnp.int32, (1, 2 * D_MODEL), 1) & 1
    half = lax.broadcasted_iota(jnp.int32, (1, 2 * D_MODEL), 1) < D_MODEL
    halfk = lax.broadcasted_iota(jnp.int32, (1, 2 * n_chroms), 1) < n_chroms
    k48 = (lax.broadcasted_iota(jnp.int32, (1, 2 * n_chroms), 1) % n_chroms
           ).astype(jnp.float32)
    ncols = 16

    def step(j, _):
        e8 = (lax.broadcasted_iota(jnp.int32, (ba, ncols), 0)
              == j * ncols + lax.broadcasted_iota(jnp.int32, (ba, ncols), 1)
              ).astype(jnp.float32)
        p8 = lax.dot_general(pos_blk, e8, (((1,), (0,)), ((), ())),
                             precision=lax.Precision.HIGHEST,
                             preferred_element_type=jnp.float32)
        c8 = lax.dot_general(cidf_blk, e8, (((1,), (0,)), ((), ())),
                             preferred_element_type=jnp.float32)
        for g in range(ncols // 2):
            p0 = p8[:, 2 * g:2 * g + 1]
            p1 = p8[:, 2 * g + 1:2 * g + 2]
            p2 = jnp.where(half, p0, p1)
            ang2 = p2 * freqs2
            c0 = c8[:, 2 * g:2 * g + 1]
            c1 = c8[:, 2 * g + 1:2 * g + 2]
            csel = jnp.where(halfk, c0, c1)
            oh2 = (csel == k48).astype(jnp.float32)
            emb2 = lax.dot_general(oh2, tab2, (((1,), (0,)), ((), ())),
                                   preferred_element_type=jnp.float32)
            o2 = emb2 + _sincos_lanes(ang2, parity)
            a_idx = j * ncols + 2 * g
            out_ref[pl.ds(a_idx, 1)] = o2[:, :D_MODEL][None]
            out_ref[pl.ds(a_idx + 1, 1)] = o2[:, D_MODEL:][None]
        return 0

    lax.fori_loop(0, ba // ncols, step, 0)


def _tc_encode(pos_t, cid_t, freqs2, tab2, ba):
    ll, aa = pos_t.shape
    n_chroms = tab2.shape[0] // 2
    assert aa % ba == 0
    grid = (aa // ba,)
    return pl.pallas_call(
        functools.partial(_encode_body, ba, n_chroms),
        grid=grid,
        in_specs=[
            pl.BlockSpec((ll, ba), lambda i: (0, i)),
            pl.BlockSpec((ll, ba), lambda i: (0, i)),
            pl.BlockSpec((1, 2 * D_MODEL), lambda i: (0, 0)),
            pl.BlockSpec((2 * n_chroms, 2 * D_MODEL), lambda i: (0, 0)),
        ],
        out_specs=pl.BlockSpec((ba, ll, D_MODEL), lambda i: (i, 0, 0)),
        out_shape=jax.ShapeDtypeStruct((aa, ll, D_MODEL), jnp.float32),
    )(pos_t, cid_t, freqs2, tab2)


def _freqs_row():
    i = jnp.arange(D_MODEL)
    return (jnp.asarray(MIN_FREQ, jnp.float32)
            ** (2.0 * (i // 2).astype(jnp.float32) / D_MODEL)).reshape(1, D_MODEL)


def kernel(X, positions_table, chroms, chrom_table):
    a, l = X.shape
    n = a * l
    xf = X.T.reshape(n).astype(jnp.int32)
    pos_flat, cid_flat = _make_sc_gather(n)(
        xf, positions_table.astype(jnp.float32), chroms.astype(jnp.int32))
    pos_t = pos_flat.reshape(l, a)
    cid_t = cid_flat.reshape(l, a)
    fr = _freqs_row()
    freqs2 = jnp.concatenate([fr, fr], axis=1)
    nc, d = chrom_table.shape
    z = jnp.zeros((nc, d), jnp.float32)
    tab2 = jnp.concatenate(
        [jnp.concatenate([chrom_table, z], axis=1),
         jnp.concatenate([z, chrom_table], axis=1)], axis=0)
    out = _tc_encode(pos_t, cid_t, freqs2, tab2, ba=128)
    return out
